# baseline (speedup 1.0000x reference)
import functools

import jax
import jax.numpy as jnp
from jax.experimental import pallas as pl
from jax.experimental.pallas import tpu as pltpu

_TEMPERATURE = 0.07


def _mm_kernel(t_ref, ft_ref, out_ref):
    cb = t_ref.shape[0]
    bs = ft_ref.shape[1]
    y = jnp.dot(t_ref[...], ft_ref[...], preferred_element_type=jnp.float32)
    z = y.reshape(cb, bs // 128, 128)
    out_ref[:, :, 0, :] = z
    out_ref[:, :, 1, :] = -z


@functools.partial(jax.jit, static_argnames=("interpret",))
def _run(features, text_features, interpret=False):
    bs, k = features.shape
    nc = text_features.shape[0]
    t_scaled = text_features / (2.0 * _TEMPERATURE)
    feat_t = features.T

    cb = 40
    nbt = 4
    bb = bs // nbt
    out = pl.pallas_call(
        _mm_kernel,
        grid=(nc // cb, nbt),
        in_specs=[
            pl.BlockSpec((cb, k), lambda i, j: (i, 0)),
            pl.BlockSpec((k, bb), lambda i, j: (0, j)),
        ],
        out_specs=pl.BlockSpec((cb, bb // 128, 2, 128), lambda i, j: (i, j, 0, 0)),
        out_shape=jax.ShapeDtypeStruct((nc, bs // 128, 2, 128), jnp.float32),
        compiler_params=pltpu.CompilerParams(
            dimension_semantics=("parallel", "parallel"),
        ),
        interpret=interpret,
    )(t_scaled, feat_t)
    sm = out.transpose(1, 3, 0, 2).reshape(bs, nc, 2)
    return sm


def kernel(features, text_features, targets, dataset):
    sim_matrix = _run(features, text_features)
    loss = jnp.zeros((), dtype=jnp.float32)
    return (loss, sim_matrix)


# speedup vs baseline: 1.2500x; 1.2500x over previous
import functools

import jax
import jax.numpy as jnp
from jax.experimental import pallas as pl
from jax.experimental.pallas import tpu as pltpu

_TEMPERATURE = 0.07


def _mm_kernel(t_ref, ft_ref, out_ref):
    cb = t_ref.shape[0]
    bs = ft_ref.shape[1]
    y = jnp.dot(t_ref[...], ft_ref[...], preferred_element_type=jnp.float32)
    z = y.reshape(cb, bs // 128, 128)
    out_ref[:, :, 0, :] = z
    out_ref[:, :, 1, :] = -z


@functools.partial(jax.jit, static_argnames=("interpret",))
def _run(features, text_features, interpret=False):
    bs, k = features.shape
    nc = text_features.shape[0]
    t_scaled = text_features / (2.0 * _TEMPERATURE)
    feat_t = features.T

    cb = 40
    out = pl.pallas_call(
        _mm_kernel,
        grid=(nc // cb,),
        in_specs=[
            pl.BlockSpec((cb, k), lambda i: (i, 0)),
            pl.BlockSpec((k, bs), lambda i: (0, 0)),
        ],
        out_specs=pl.BlockSpec((cb, bs // 128, 2, 128), lambda i: (i, 0, 0, 0)),
        out_shape=jax.ShapeDtypeStruct((nc, bs // 128, 2, 128), jnp.float32),
        compiler_params=pltpu.CompilerParams(
            dimension_semantics=("parallel",),
        ),
        interpret=interpret,
    )(t_scaled, feat_t)
    sm = out.transpose(1, 3, 0, 2).reshape(bs, nc, 2)
    return sm


def kernel(features, text_features, targets, dataset):
    sim_matrix = _run(features, text_features)
    loss = jnp.zeros((), dtype=jnp.float32)
    return (loss, sim_matrix)
